# baseline (device time: 1624607 ns/iter reference)
import jax
import jax.numpy as jnp
from jax import lax
from jax.experimental import pallas as pl
from jax.experimental.pallas import tpu as pltpu

N_DEV = 32
B = 2
S = 256
H = 8
D = 64
SCALE = D ** -0.5

_CompilerParams = getattr(pltpu, "CompilerParams", None) or getattr(
    pltpu, "TPUCompilerParams"
)


def kernel(Q, K, V):
    Qt = jnp.transpose(Q, (0, 2, 1, 3))
    Kt = jnp.transpose(K, (0, 2, 1, 3))
    Vt = jnp.transpose(V, (0, 2, 1, 3))

    def body(q_ref, k_ref, v_ref, out_ref, k_buf, v_buf, l_ref, acc_ref,
             send_sems, recv_sems, credit_sem):
        my = lax.axis_index("i")
        left = lax.rem(my + N_DEV - 1, N_DEV)
        right = lax.rem(my + 1, N_DEV)

        barrier_sem = pltpu.get_barrier_semaphore()
        for nbr in (left, right):
            pl.semaphore_signal(
                barrier_sem, inc=1,
                device_id=(nbr,), device_id_type=pl.DeviceIdType.MESH,
            )
        pl.semaphore_wait(barrier_sem, 2)

        l_ref[...] = jnp.zeros_like(l_ref)
        acc_ref[...] = jnp.zeros_like(acc_ref)
        k_buf[0] = k_ref[...]
        v_buf[0] = v_ref[...]

        def accumulate(load_k, load_v):
            for b in range(B):
                for hh in range(H):
                    q = q_ref[b, hh] * SCALE
                    k = load_k(b, hh)
                    v = load_v(b, hh)
                    s = lax.dot_general(
                        q, k, (((1,), (1,)), ((), ())),
                        preferred_element_type=jnp.float32,
                    )
                    p = jnp.exp(s)
                    l_ref[b, hh] += jnp.broadcast_to(
                        jnp.sum(p, axis=1, keepdims=True), (S, D)
                    )
                    acc_ref[b, hh] += lax.dot_general(
                        p, v, (((1,), (0,)), ((), ())),
                        preferred_element_type=jnp.float32,
                    )

        accumulate(lambda b, hh: k_ref[b, hh], lambda b, hh: v_ref[b, hh])

        def hop(h, carry):
            send_slot = lax.rem(h, 2)
            recv_slot = lax.rem(h + 1, 2)

            @pl.when(h >= 1)
            def _():
                pl.semaphore_wait(credit_sem, 1)

            rk = pltpu.make_async_remote_copy(
                src_ref=k_buf.at[send_slot],
                dst_ref=k_buf.at[recv_slot],
                send_sem=send_sems.at[0, send_slot],
                recv_sem=recv_sems.at[0, recv_slot],
                device_id=(right,),
                device_id_type=pl.DeviceIdType.MESH,
            )
            rv = pltpu.make_async_remote_copy(
                src_ref=v_buf.at[send_slot],
                dst_ref=v_buf.at[recv_slot],
                send_sem=send_sems.at[1, send_slot],
                recv_sem=recv_sems.at[1, recv_slot],
                device_id=(right,),
                device_id_type=pl.DeviceIdType.MESH,
            )
            rk.start()
            rv.start()
            rk.wait()
            rv.wait()

            @pl.when(h <= N_DEV - 3)
            def _():
                pl.semaphore_signal(
                    credit_sem, inc=1,
                    device_id=(left,), device_id_type=pl.DeviceIdType.MESH,
                )

            accumulate(
                lambda b, hh: k_buf[recv_slot, b, hh],
                lambda b, hh: v_buf[recv_slot, b, hh],
            )
            return carry

        lax.fori_loop(0, N_DEV - 1, hop, 0)

        for b in range(B):
            for hh in range(H):
                out_ref[b, hh] = acc_ref[b, hh] / l_ref[b, hh]

    out_t = pl.pallas_call(
        body,
        out_shape=jax.ShapeDtypeStruct((B, H, S, D), jnp.float32),
        in_specs=[pl.BlockSpec(memory_space=pltpu.VMEM)] * 3,
        out_specs=pl.BlockSpec(memory_space=pltpu.VMEM),
        scratch_shapes=[
            pltpu.VMEM((2, B, H, S, D), jnp.float32),
            pltpu.VMEM((2, B, H, S, D), jnp.float32),
            pltpu.VMEM((B, H, S, D), jnp.float32),
            pltpu.VMEM((B, H, S, D), jnp.float32),
            pltpu.SemaphoreType.DMA((2, 2)),
            pltpu.SemaphoreType.DMA((2, 2)),
            pltpu.SemaphoreType.REGULAR,
        ],
        compiler_params=_CompilerParams(collective_id=0),
    )(Qt, Kt, Vt)

    return jnp.transpose(out_t, (0, 2, 1, 3))


# device time: 1486623 ns/iter; 1.0928x vs baseline; 1.0928x over previous
import jax
import jax.numpy as jnp
from jax import lax
from jax.experimental import pallas as pl
from jax.experimental.pallas import tpu as pltpu

N_DEV = 32
NSTEP = N_DEV // 2
B = 2
S = 256
H = 8
D = 64
SCALE = D ** -0.5

_CompilerParams = getattr(pltpu, "CompilerParams", None) or getattr(
    pltpu, "TPUCompilerParams"
)


def kernel(Q, K, V):
    Qt = jnp.transpose(Q, (0, 2, 1, 3))
    Kt = jnp.transpose(K, (0, 2, 1, 3))
    Vt = jnp.transpose(V, (0, 2, 1, 3))

    def body(q_ref, k_ref, v_ref, out_ref,
             qb_ref, kr_buf, vr_buf, kl_buf, vl_buf, l_ref, acc_ref,
             send_sems, recv_sems, credit_r, credit_l):
        my = lax.axis_index("i")
        left = lax.rem(my + N_DEV - 1, N_DEV)
        right = lax.rem(my + 1, N_DEV)

        barrier_sem = pltpu.get_barrier_semaphore()
        for nbr in (left, right):
            pl.semaphore_signal(
                barrier_sem, inc=1,
                device_id=(nbr,), device_id_type=pl.DeviceIdType.MESH,
            )
        pl.semaphore_wait(barrier_sem, 2)

        l_ref[...] = jnp.zeros_like(l_ref)
        acc_ref[...] = jnp.zeros_like(acc_ref)
        qb_ref[...] = (q_ref[...] * SCALE).astype(jnp.bfloat16)
        kr_buf[0] = k_ref[...]
        vr_buf[0] = v_ref[...]
        kl_buf[0] = k_ref[...]
        vl_buf[0] = v_ref[...]

        def accumulate(load_k, load_v):
            for b in range(B):
                for hh in range(H):
                    q = qb_ref[b, hh]
                    k = load_k(b, hh).astype(jnp.bfloat16)
                    v = load_v(b, hh).astype(jnp.bfloat16)
                    s_ = lax.dot_general(
                        q, k, (((1,), (1,)), ((), ())),
                        preferred_element_type=jnp.float32,
                    )
                    p = jnp.exp(s_)
                    l_ref[b, hh] += jnp.broadcast_to(
                        jnp.sum(p, axis=1, keepdims=True), (S, D)
                    )
                    acc_ref[b, hh] += lax.dot_general(
                        p.astype(jnp.bfloat16), v, (((1,), (0,)), ((), ())),
                        preferred_element_type=jnp.float32,
                    )

        def step(s, carry):
            ssl = lax.rem(s, 2)
            rsl = lax.rem(s + 1, 2)

            @pl.when(s >= 1)
            def _():
                pl.semaphore_wait(credit_r, 1)

            @pl.when(jnp.logical_and(s >= 1, s <= NSTEP - 2))
            def _():
                pl.semaphore_wait(credit_l, 1)

            rkR = pltpu.make_async_remote_copy(
                src_ref=kr_buf.at[ssl], dst_ref=kr_buf.at[rsl],
                send_sem=send_sems.at[0, 0, ssl],
                recv_sem=recv_sems.at[0, 0, rsl],
                device_id=(right,), device_id_type=pl.DeviceIdType.MESH,
            )
            rvR = pltpu.make_async_remote_copy(
                src_ref=vr_buf.at[ssl], dst_ref=vr_buf.at[rsl],
                send_sem=send_sems.at[0, 1, ssl],
                recv_sem=recv_sems.at[0, 1, rsl],
                device_id=(right,), device_id_type=pl.DeviceIdType.MESH,
            )
            rkL = pltpu.make_async_remote_copy(
                src_ref=kl_buf.at[ssl], dst_ref=kl_buf.at[rsl],
                send_sem=send_sems.at[1, 0, ssl],
                recv_sem=recv_sems.at[1, 0, rsl],
                device_id=(left,), device_id_type=pl.DeviceIdType.MESH,
            )
            rvL = pltpu.make_async_remote_copy(
                src_ref=vl_buf.at[ssl], dst_ref=vl_buf.at[rsl],
                send_sem=send_sems.at[1, 1, ssl],
                recv_sem=recv_sems.at[1, 1, rsl],
                device_id=(left,), device_id_type=pl.DeviceIdType.MESH,
            )

            rkR.start()
            rvR.start()

            @pl.when(s <= NSTEP - 2)
            def _():
                rkL.start()
                rvL.start()

            accumulate(
                lambda b, hh: kr_buf[ssl, b, hh],
                lambda b, hh: vr_buf[ssl, b, hh],
            )

            @pl.when(s >= 1)
            def _():
                accumulate(
                    lambda b, hh: kl_buf[ssl, b, hh],
                    lambda b, hh: vl_buf[ssl, b, hh],
                )

            rkR.wait()
            rvR.wait()

            @pl.when(s <= NSTEP - 2)
            def _():
                rkL.wait()
                rvL.wait()

            @pl.when(s <= NSTEP - 2)
            def _():
                pl.semaphore_signal(
                    credit_r, inc=1,
                    device_id=(left,), device_id_type=pl.DeviceIdType.MESH,
                )

            @pl.when(s <= NSTEP - 3)
            def _():
                pl.semaphore_signal(
                    credit_l, inc=1,
                    device_id=(right,), device_id_type=pl.DeviceIdType.MESH,
                )
            return carry

        lax.fori_loop(0, NSTEP, step, 0)

        accumulate(
            lambda b, hh: kr_buf[0, b, hh],
            lambda b, hh: vr_buf[0, b, hh],
        )

        for b in range(B):
            for hh in range(H):
                out_ref[b, hh] = acc_ref[b, hh] / l_ref[b, hh]

    out_t = pl.pallas_call(
        body,
        out_shape=jax.ShapeDtypeStruct((B, H, S, D), jnp.float32),
        in_specs=[pl.BlockSpec(memory_space=pltpu.VMEM)] * 3,
        out_specs=pl.BlockSpec(memory_space=pltpu.VMEM),
        scratch_shapes=[
            pltpu.VMEM((B, H, S, D), jnp.bfloat16),
            pltpu.VMEM((2, B, H, S, D), jnp.float32),
            pltpu.VMEM((2, B, H, S, D), jnp.float32),
            pltpu.VMEM((2, B, H, S, D), jnp.float32),
            pltpu.VMEM((2, B, H, S, D), jnp.float32),
            pltpu.VMEM((B, H, S, D), jnp.float32),
            pltpu.VMEM((B, H, S, D), jnp.float32),
            pltpu.SemaphoreType.DMA((2, 2, 2)),
            pltpu.SemaphoreType.DMA((2, 2, 2)),
            pltpu.SemaphoreType.REGULAR,
            pltpu.SemaphoreType.REGULAR,
        ],
        compiler_params=_CompilerParams(collective_id=0),
    )(Qt, Kt, Vt)

    return jnp.transpose(out_t, (0, 2, 1, 3))


# device time: 771274 ns/iter; 2.1064x vs baseline; 1.9275x over previous
import jax
import jax.numpy as jnp
from jax import lax
from jax.experimental import pallas as pl
from jax.experimental.pallas import tpu as pltpu

N_DEV = 32


def _lid(x, y, z):
    return z * 8 + y * 2 + (x if y % 2 == 0 else 1 - x)


_P44 = [
    (y, z)
    for z in range(4)
    for y in (range(4) if z % 2 == 0 else range(3, -1, -1))
]
_CYCLE = [(0, y, z) for (y, z) in _P44] + [(1, y, z) for (y, z) in reversed(_P44)]
assert all(
    sum(abs(a - b) for a, b in zip(_CYCLE[i], _CYCLE[(i + 1) % 32])) == 1
    for i in range(32)
), "cycle edge is not a single hop"
_RING = [_lid(x, y, z) for (x, y, z) in _CYCLE]
assert sorted(_RING) == list(range(32))
def _ring_neighbors(my):
    z = my // 8
    r = my - 8 * z
    y = r // 2
    xb = r - 2 * y
    x = jnp.where(y % 2 == 0, xb, 1 - xb)
    t = z * 4 + jnp.where(z % 2 == 0, y, 3 - y)

    def coords_from(x_, t_):
        z_ = t_ // 4
        u = t_ - 4 * z_
        y_ = jnp.where(z_ % 2 == 0, u, 3 - u)
        return x_, y_, z_

    def lid(x_, y_, z_):
        return z_ * 8 + y_ * 2 + jnp.where(y_ % 2 == 0, x_, 1 - x_)

    sx = jnp.where(x == 0, jnp.where(t == 15, 1, 0), jnp.where(t == 0, 0, 1))
    st = jnp.where(x == 0, jnp.where(t == 15, 15, t + 1),
                   jnp.where(t == 0, 0, t - 1))
    right = lid(*coords_from(sx, st))
    px = jnp.where(x == 0, jnp.where(t == 0, 1, 0), jnp.where(t == 15, 0, 1))
    pt = jnp.where(x == 0, jnp.where(t == 0, 0, t - 1),
                   jnp.where(t == 15, 15, t + 1))
    left = lid(*coords_from(px, pt))
    return left, right
NSTEP = N_DEV // 2
B = 2
S = 256
H = 8
D = 64
SCALE = D ** -0.5

_CompilerParams = getattr(pltpu, "CompilerParams", None) or getattr(
    pltpu, "TPUCompilerParams"
)


def kernel(Q, K, V):
    Qt = jnp.transpose(Q, (0, 2, 1, 3))
    Kt = jnp.transpose(K, (0, 2, 1, 3))
    Vt = jnp.transpose(V, (0, 2, 1, 3))

    def body(q_ref, k_ref, v_ref, out_ref,
             qb_ref, kr_buf, vr_buf, kl_buf, vl_buf, l_ref, acc_ref,
             send_sems, recv_sems, credit_r, credit_l):
        my = lax.axis_index("i")
        left, right = _ring_neighbors(my)

        barrier_sem = pltpu.get_barrier_semaphore()
        for nbr in (left, right):
            pl.semaphore_signal(
                barrier_sem, inc=1,
                device_id=(nbr,), device_id_type=pl.DeviceIdType.MESH,
            )
        pl.semaphore_wait(barrier_sem, 2)

        l_ref[...] = jnp.zeros_like(l_ref)
        acc_ref[...] = jnp.zeros_like(acc_ref)
        qb_ref[...] = (q_ref[...] * SCALE).astype(jnp.bfloat16)
        kr_buf[0] = k_ref[...]
        vr_buf[0] = v_ref[...]
        kl_buf[0] = k_ref[...]
        vl_buf[0] = v_ref[...]

        def accumulate(load_k, load_v):
            for b in range(B):
                for hh in range(H):
                    q = qb_ref[b, hh]
                    k = load_k(b, hh).astype(jnp.bfloat16)
                    v = load_v(b, hh).astype(jnp.bfloat16)
                    s_ = lax.dot_general(
                        q, k, (((1,), (1,)), ((), ())),
                        preferred_element_type=jnp.float32,
                    )
                    p = jnp.exp(s_)
                    l_ref[b, hh] += jnp.broadcast_to(
                        jnp.sum(p, axis=1, keepdims=True), (S, D)
                    )
                    acc_ref[b, hh] += lax.dot_general(
                        p.astype(jnp.bfloat16), v, (((1,), (0,)), ((), ())),
                        preferred_element_type=jnp.float32,
                    )

        def step(s, carry):
            ssl = lax.rem(s, 2)
            rsl = lax.rem(s + 1, 2)

            @pl.when(s >= 1)
            def _():
                pl.semaphore_wait(credit_r, 1)

            @pl.when(jnp.logical_and(s >= 1, s <= NSTEP - 2))
            def _():
                pl.semaphore_wait(credit_l, 1)

            rkR = pltpu.make_async_remote_copy(
                src_ref=kr_buf.at[ssl], dst_ref=kr_buf.at[rsl],
                send_sem=send_sems.at[0, 0, ssl],
                recv_sem=recv_sems.at[0, 0, rsl],
                device_id=(right,), device_id_type=pl.DeviceIdType.MESH,
            )
            rvR = pltpu.make_async_remote_copy(
                src_ref=vr_buf.at[ssl], dst_ref=vr_buf.at[rsl],
                send_sem=send_sems.at[0, 1, ssl],
                recv_sem=recv_sems.at[0, 1, rsl],
                device_id=(right,), device_id_type=pl.DeviceIdType.MESH,
            )
            rkL = pltpu.make_async_remote_copy(
                src_ref=kl_buf.at[ssl], dst_ref=kl_buf.at[rsl],
                send_sem=send_sems.at[1, 0, ssl],
                recv_sem=recv_sems.at[1, 0, rsl],
                device_id=(left,), device_id_type=pl.DeviceIdType.MESH,
            )
            rvL = pltpu.make_async_remote_copy(
                src_ref=vl_buf.at[ssl], dst_ref=vl_buf.at[rsl],
                send_sem=send_sems.at[1, 1, ssl],
                recv_sem=recv_sems.at[1, 1, rsl],
                device_id=(left,), device_id_type=pl.DeviceIdType.MESH,
            )

            rkR.start()
            rvR.start()

            @pl.when(s <= NSTEP - 2)
            def _():
                rkL.start()
                rvL.start()

            accumulate(
                lambda b, hh: kr_buf[ssl, b, hh],
                lambda b, hh: vr_buf[ssl, b, hh],
            )

            @pl.when(s >= 1)
            def _():
                accumulate(
                    lambda b, hh: kl_buf[ssl, b, hh],
                    lambda b, hh: vl_buf[ssl, b, hh],
                )

            rkR.wait()
            rvR.wait()

            @pl.when(s <= NSTEP - 2)
            def _():
                rkL.wait()
                rvL.wait()

            @pl.when(s <= NSTEP - 2)
            def _():
                pl.semaphore_signal(
                    credit_r, inc=1,
                    device_id=(left,), device_id_type=pl.DeviceIdType.MESH,
                )

            @pl.when(s <= NSTEP - 3)
            def _():
                pl.semaphore_signal(
                    credit_l, inc=1,
                    device_id=(right,), device_id_type=pl.DeviceIdType.MESH,
                )
            return carry

        lax.fori_loop(0, NSTEP, step, 0)

        accumulate(
            lambda b, hh: kr_buf[0, b, hh],
            lambda b, hh: vr_buf[0, b, hh],
        )

        for b in range(B):
            for hh in range(H):
                out_ref[b, hh] = acc_ref[b, hh] / l_ref[b, hh]

    out_t = pl.pallas_call(
        body,
        out_shape=jax.ShapeDtypeStruct((B, H, S, D), jnp.float32),
        in_specs=[pl.BlockSpec(memory_space=pltpu.VMEM)] * 3,
        out_specs=pl.BlockSpec(memory_space=pltpu.VMEM),
        scratch_shapes=[
            pltpu.VMEM((B, H, S, D), jnp.bfloat16),
            pltpu.VMEM((2, B, H, S, D), jnp.float32),
            pltpu.VMEM((2, B, H, S, D), jnp.float32),
            pltpu.VMEM((2, B, H, S, D), jnp.float32),
            pltpu.VMEM((2, B, H, S, D), jnp.float32),
            pltpu.VMEM((B, H, S, D), jnp.float32),
            pltpu.VMEM((B, H, S, D), jnp.float32),
            pltpu.SemaphoreType.DMA((2, 2, 2)),
            pltpu.SemaphoreType.DMA((2, 2, 2)),
            pltpu.SemaphoreType.REGULAR,
            pltpu.SemaphoreType.REGULAR,
        ],
        compiler_params=_CompilerParams(collective_id=0),
    )(Qt, Kt, Vt)

    return jnp.transpose(out_t, (0, 2, 1, 3))


# device time: 411262 ns/iter; 3.9503x vs baseline; 1.8754x over previous
import jax
import jax.numpy as jnp
from jax import lax
from jax.experimental import pallas as pl
from jax.experimental.pallas import tpu as pltpu

N_DEV = 32


def _lid(x, y, z):
    return z * 8 + y * 2 + (x if y % 2 == 0 else 1 - x)


_P44 = [
    (y, z)
    for z in range(4)
    for y in (range(4) if z % 2 == 0 else range(3, -1, -1))
]
_CYCLE = [(0, y, z) for (y, z) in _P44] + [(1, y, z) for (y, z) in reversed(_P44)]
assert all(
    sum(abs(a - b) for a, b in zip(_CYCLE[i], _CYCLE[(i + 1) % 32])) == 1
    for i in range(32)
), "cycle edge is not a single hop"
_RING = [_lid(x, y, z) for (x, y, z) in _CYCLE]
assert sorted(_RING) == list(range(32))
def _ring_neighbors(my):
    z = my // 8
    r = my - 8 * z
    y = r // 2
    xb = r - 2 * y
    x = jnp.where(y % 2 == 0, xb, 1 - xb)
    t = z * 4 + jnp.where(z % 2 == 0, y, 3 - y)

    def coords_from(x_, t_):
        z_ = t_ // 4
        u = t_ - 4 * z_
        y_ = jnp.where(z_ % 2 == 0, u, 3 - u)
        return x_, y_, z_

    def lid(x_, y_, z_):
        return z_ * 8 + y_ * 2 + jnp.where(y_ % 2 == 0, x_, 1 - x_)

    sx = jnp.where(x == 0, jnp.where(t == 15, 1, 0), jnp.where(t == 0, 0, 1))
    st = jnp.where(x == 0, jnp.where(t == 15, 15, t + 1),
                   jnp.where(t == 0, 0, t - 1))
    right = lid(*coords_from(sx, st))
    px = jnp.where(x == 0, jnp.where(t == 0, 1, 0), jnp.where(t == 15, 0, 1))
    pt = jnp.where(x == 0, jnp.where(t == 0, 0, t - 1),
                   jnp.where(t == 15, 15, t + 1))
    left = lid(*coords_from(px, pt))
    return left, right
NSTEP = N_DEV // 2
B = 2
S = 256
H = 8
D = 64
SCALE = D ** -0.5

_CompilerParams = getattr(pltpu, "CompilerParams", None) or getattr(
    pltpu, "TPUCompilerParams"
)


def kernel(Q, K, V):
    Qt = jnp.transpose(Q, (0, 2, 1, 3))
    Kt = jnp.transpose(K, (0, 2, 1, 3)).astype(jnp.bfloat16)
    Vt = jnp.transpose(V, (0, 2, 1, 3)).astype(jnp.bfloat16)

    def body(q_ref, k_ref, v_ref, out_ref,
             qb_ref, kr_buf, vr_buf, kl_buf, vl_buf, l_ref, acc_ref,
             send_sems, recv_sems, credit_r, credit_l):
        my = lax.axis_index("i")
        left, right = _ring_neighbors(my)

        barrier_sem = pltpu.get_barrier_semaphore()
        for nbr in (left, right):
            pl.semaphore_signal(
                barrier_sem, inc=1,
                device_id=(nbr,), device_id_type=pl.DeviceIdType.MESH,
            )
        pl.semaphore_wait(barrier_sem, 2)

        l_ref[...] = jnp.zeros_like(l_ref)
        acc_ref[...] = jnp.zeros_like(acc_ref)
        qb_ref[...] = (q_ref[...] * SCALE).astype(jnp.bfloat16)
        kr_buf[0] = k_ref[...]
        vr_buf[0] = v_ref[...]
        kl_buf[0] = k_ref[...]
        vl_buf[0] = v_ref[...]

        def accumulate(load_k, load_v):
            for b in range(B):
                for hh in range(H):
                    q = qb_ref[b, hh]
                    k = load_k(b, hh)
                    v = load_v(b, hh)
                    s_ = lax.dot_general(
                        q, k, (((1,), (1,)), ((), ())),
                        preferred_element_type=jnp.float32,
                    )
                    p = jnp.exp(s_)
                    l_ref[b, hh] += jnp.broadcast_to(
                        jnp.sum(p, axis=1, keepdims=True), (S, D)
                    )
                    acc_ref[b, hh] += lax.dot_general(
                        p.astype(jnp.bfloat16), v, (((1,), (0,)), ((), ())),
                        preferred_element_type=jnp.float32,
                    )

        def step(s, carry):
            ssl = lax.rem(s, 2)
            rsl = lax.rem(s + 1, 2)

            @pl.when(s >= 1)
            def _():
                pl.semaphore_wait(credit_r, 1)

            @pl.when(jnp.logical_and(s >= 1, s <= NSTEP - 2))
            def _():
                pl.semaphore_wait(credit_l, 1)

            rkR = pltpu.make_async_remote_copy(
                src_ref=kr_buf.at[ssl], dst_ref=kr_buf.at[rsl],
                send_sem=send_sems.at[0, 0, ssl],
                recv_sem=recv_sems.at[0, 0, rsl],
                device_id=(right,), device_id_type=pl.DeviceIdType.MESH,
            )
            rvR = pltpu.make_async_remote_copy(
                src_ref=vr_buf.at[ssl], dst_ref=vr_buf.at[rsl],
                send_sem=send_sems.at[0, 1, ssl],
                recv_sem=recv_sems.at[0, 1, rsl],
                device_id=(right,), device_id_type=pl.DeviceIdType.MESH,
            )
            rkL = pltpu.make_async_remote_copy(
                src_ref=kl_buf.at[ssl], dst_ref=kl_buf.at[rsl],
                send_sem=send_sems.at[1, 0, ssl],
                recv_sem=recv_sems.at[1, 0, rsl],
                device_id=(left,), device_id_type=pl.DeviceIdType.MESH,
            )
            rvL = pltpu.make_async_remote_copy(
                src_ref=vl_buf.at[ssl], dst_ref=vl_buf.at[rsl],
                send_sem=send_sems.at[1, 1, ssl],
                recv_sem=recv_sems.at[1, 1, rsl],
                device_id=(left,), device_id_type=pl.DeviceIdType.MESH,
            )

            rkR.start()
            rvR.start()

            @pl.when(s <= NSTEP - 2)
            def _():
                rkL.start()
                rvL.start()

            accumulate(
                lambda b, hh: kr_buf[ssl, b, hh],
                lambda b, hh: vr_buf[ssl, b, hh],
            )

            @pl.when(s >= 1)
            def _():
                accumulate(
                    lambda b, hh: kl_buf[ssl, b, hh],
                    lambda b, hh: vl_buf[ssl, b, hh],
                )

            rkR.wait()
            rvR.wait()

            @pl.when(s <= NSTEP - 2)
            def _():
                rkL.wait()
                rvL.wait()

            @pl.when(s <= NSTEP - 2)
            def _():
                pl.semaphore_signal(
                    credit_r, inc=1,
                    device_id=(left,), device_id_type=pl.DeviceIdType.MESH,
                )

            @pl.when(s <= NSTEP - 3)
            def _():
                pl.semaphore_signal(
                    credit_l, inc=1,
                    device_id=(right,), device_id_type=pl.DeviceIdType.MESH,
                )
            return carry

        lax.fori_loop(0, NSTEP, step, 0)

        accumulate(
            lambda b, hh: kr_buf[0, b, hh],
            lambda b, hh: vr_buf[0, b, hh],
        )

        for b in range(B):
            for hh in range(H):
                out_ref[b, hh] = acc_ref[b, hh] / l_ref[b, hh]

    out_t = pl.pallas_call(
        body,
        out_shape=jax.ShapeDtypeStruct((B, H, S, D), jnp.float32),
        in_specs=[pl.BlockSpec(memory_space=pltpu.VMEM)] * 3,
        out_specs=pl.BlockSpec(memory_space=pltpu.VMEM),
        scratch_shapes=[
            pltpu.VMEM((B, H, S, D), jnp.bfloat16),
            pltpu.VMEM((2, B, H, S, D), jnp.bfloat16),
            pltpu.VMEM((2, B, H, S, D), jnp.bfloat16),
            pltpu.VMEM((2, B, H, S, D), jnp.bfloat16),
            pltpu.VMEM((2, B, H, S, D), jnp.bfloat16),
            pltpu.VMEM((B, H, S, D), jnp.float32),
            pltpu.VMEM((B, H, S, D), jnp.float32),
            pltpu.SemaphoreType.DMA((2, 2, 2)),
            pltpu.SemaphoreType.DMA((2, 2, 2)),
            pltpu.SemaphoreType.REGULAR,
            pltpu.SemaphoreType.REGULAR,
        ],
        compiler_params=_CompilerParams(collective_id=0),
    )(Qt, Kt, Vt)

    return jnp.transpose(out_t, (0, 2, 1, 3))


# device time: 410726 ns/iter; 3.9555x vs baseline; 1.0013x over previous
import jax
import jax.numpy as jnp
from jax import lax
from jax.experimental import pallas as pl
from jax.experimental.pallas import tpu as pltpu

N_DEV = 32


def _lid(x, y, z):
    return z * 8 + y * 2 + (x if y % 2 == 0 else 1 - x)


_P44 = [
    (y, z)
    for z in range(4)
    for y in (range(4) if z % 2 == 0 else range(3, -1, -1))
]
_CYCLE = [(0, y, z) for (y, z) in _P44] + [(1, y, z) for (y, z) in reversed(_P44)]
assert all(
    sum(abs(a - b) for a, b in zip(_CYCLE[i], _CYCLE[(i + 1) % 32])) == 1
    for i in range(32)
), "cycle edge is not a single hop"
_RING = [_lid(x, y, z) for (x, y, z) in _CYCLE]
assert sorted(_RING) == list(range(32))
def _ring_neighbors(my):
    z = my // 8
    r = my - 8 * z
    y = r // 2
    xb = r - 2 * y
    x = jnp.where(y % 2 == 0, xb, 1 - xb)
    t = z * 4 + jnp.where(z % 2 == 0, y, 3 - y)

    def coords_from(x_, t_):
        z_ = t_ // 4
        u = t_ - 4 * z_
        y_ = jnp.where(z_ % 2 == 0, u, 3 - u)
        return x_, y_, z_

    def lid(x_, y_, z_):
        return z_ * 8 + y_ * 2 + jnp.where(y_ % 2 == 0, x_, 1 - x_)

    sx = jnp.where(x == 0, jnp.where(t == 15, 1, 0), jnp.where(t == 0, 0, 1))
    st = jnp.where(x == 0, jnp.where(t == 15, 15, t + 1),
                   jnp.where(t == 0, 0, t - 1))
    right = lid(*coords_from(sx, st))
    px = jnp.where(x == 0, jnp.where(t == 0, 1, 0), jnp.where(t == 15, 0, 1))
    pt = jnp.where(x == 0, jnp.where(t == 0, 0, t - 1),
                   jnp.where(t == 15, 15, t + 1))
    left = lid(*coords_from(px, pt))
    return left, right
NSTEP = N_DEV // 2
B = 2
S = 256
H = 8
D = 64
SCALE = D ** -0.5

_CompilerParams = getattr(pltpu, "CompilerParams", None) or getattr(
    pltpu, "TPUCompilerParams"
)


def kernel(Q, K, V):
    Qt = jnp.transpose(Q, (0, 2, 1, 3))
    Kt = jnp.transpose(K, (0, 2, 1, 3)).astype(jnp.bfloat16)
    Vt = jnp.transpose(V, (0, 2, 1, 3)).astype(jnp.bfloat16)

    def body(q_ref, k_ref, v_ref, out_ref,
             qb_ref, kr_buf, vr_buf, kl_buf, vl_buf, acc_ref,
             send_sems, recv_sems, credit_r, credit_l):
        my = lax.axis_index("i")
        left, right = _ring_neighbors(my)

        barrier_sem = pltpu.get_barrier_semaphore()
        for nbr in (left, right):
            pl.semaphore_signal(
                barrier_sem, inc=1,
                device_id=(nbr,), device_id_type=pl.DeviceIdType.MESH,
            )
        pl.semaphore_wait(barrier_sem, 2)

        acc_ref[...] = jnp.zeros_like(acc_ref)
        qb_ref[...] = (q_ref[...] * SCALE).astype(jnp.bfloat16)
        kr_buf[0] = k_ref[...]
        vr_buf[0] = v_ref[...]
        kl_buf[0] = k_ref[...]
        vl_buf[0] = v_ref[...]

        ones_bf = jnp.ones((S, D), jnp.bfloat16)

        def accumulate(load_k, load_v):
            for b in range(B):
                for hh in range(H):
                    q = qb_ref[b, hh]
                    k = load_k(b, hh)
                    v = load_v(b, hh)
                    s_ = lax.dot_general(
                        q, k, (((1,), (1,)), ((), ())),
                        preferred_element_type=jnp.float32,
                    )
                    p = jnp.exp(s_).astype(jnp.bfloat16)
                    vcat = jnp.concatenate([v, ones_bf], axis=1)
                    acc_ref[b, hh] += lax.dot_general(
                        p, vcat, (((1,), (0,)), ((), ())),
                        preferred_element_type=jnp.float32,
                    )

        def step(s, carry):
            ssl = lax.rem(s, 2)
            rsl = lax.rem(s + 1, 2)

            @pl.when(s >= 1)
            def _():
                pl.semaphore_wait(credit_r, 1)

            @pl.when(jnp.logical_and(s >= 1, s <= NSTEP - 2))
            def _():
                pl.semaphore_wait(credit_l, 1)

            rkR = pltpu.make_async_remote_copy(
                src_ref=kr_buf.at[ssl], dst_ref=kr_buf.at[rsl],
                send_sem=send_sems.at[0, 0, ssl],
                recv_sem=recv_sems.at[0, 0, rsl],
                device_id=(right,), device_id_type=pl.DeviceIdType.MESH,
            )
            rvR = pltpu.make_async_remote_copy(
                src_ref=vr_buf.at[ssl], dst_ref=vr_buf.at[rsl],
                send_sem=send_sems.at[0, 1, ssl],
                recv_sem=recv_sems.at[0, 1, rsl],
                device_id=(right,), device_id_type=pl.DeviceIdType.MESH,
            )
            rkL = pltpu.make_async_remote_copy(
                src_ref=kl_buf.at[ssl], dst_ref=kl_buf.at[rsl],
                send_sem=send_sems.at[1, 0, ssl],
                recv_sem=recv_sems.at[1, 0, rsl],
                device_id=(left,), device_id_type=pl.DeviceIdType.MESH,
            )
            rvL = pltpu.make_async_remote_copy(
                src_ref=vl_buf.at[ssl], dst_ref=vl_buf.at[rsl],
                send_sem=send_sems.at[1, 1, ssl],
                recv_sem=recv_sems.at[1, 1, rsl],
                device_id=(left,), device_id_type=pl.DeviceIdType.MESH,
            )

            rkR.start()
            rvR.start()

            @pl.when(s <= NSTEP - 2)
            def _():
                rkL.start()
                rvL.start()

            accumulate(
                lambda b, hh: kr_buf[ssl, b, hh],
                lambda b, hh: vr_buf[ssl, b, hh],
            )

            @pl.when(s >= 1)
            def _():
                accumulate(
                    lambda b, hh: kl_buf[ssl, b, hh],
                    lambda b, hh: vl_buf[ssl, b, hh],
                )

            rkR.wait()
            rvR.wait()

            @pl.when(s <= NSTEP - 2)
            def _():
                rkL.wait()
                rvL.wait()

            @pl.when(s <= NSTEP - 2)
            def _():
                pl.semaphore_signal(
                    credit_r, inc=1,
                    device_id=(left,), device_id_type=pl.DeviceIdType.MESH,
                )

            @pl.when(s <= NSTEP - 3)
            def _():
                pl.semaphore_signal(
                    credit_l, inc=1,
                    device_id=(right,), device_id_type=pl.DeviceIdType.MESH,
                )
            return carry

        lax.fori_loop(0, NSTEP, step, 0)

        accumulate(
            lambda b, hh: kr_buf[0, b, hh],
            lambda b, hh: vr_buf[0, b, hh],
        )

        for b in range(B):
            for hh in range(H):
                a = acc_ref[b, hh]
                out_ref[b, hh] = a[:, :D] / a[:, D:]

    out_t = pl.pallas_call(
        body,
        out_shape=jax.ShapeDtypeStruct((B, H, S, D), jnp.float32),
        in_specs=[pl.BlockSpec(memory_space=pltpu.VMEM)] * 3,
        out_specs=pl.BlockSpec(memory_space=pltpu.VMEM),
        scratch_shapes=[
            pltpu.VMEM((B, H, S, D), jnp.bfloat16),
            pltpu.VMEM((2, B, H, S, D), jnp.bfloat16),
            pltpu.VMEM((2, B, H, S, D), jnp.bfloat16),
            pltpu.VMEM((2, B, H, S, D), jnp.bfloat16),
            pltpu.VMEM((2, B, H, S, D), jnp.bfloat16),
            pltpu.VMEM((B, H, S, 2 * D), jnp.float32),
            pltpu.SemaphoreType.DMA((2, 2, 2)),
            pltpu.SemaphoreType.DMA((2, 2, 2)),
            pltpu.SemaphoreType.REGULAR,
            pltpu.SemaphoreType.REGULAR,
        ],
        compiler_params=_CompilerParams(collective_id=0),
    )(Qt, Kt, Vt)

    return jnp.transpose(out_t, (0, 2, 1, 3))
